# baseline (speedup 1.0000x reference)
import functools

import jax
import jax.numpy as jnp
from jax import lax
from jax.experimental import pallas as pl
from jax.experimental.pallas import tpu as pltpu
from jax.experimental.pallas import tpu_sc as plsc

BATCH = 16384
FEAT = 2048
L = 16
NC, NS = 2, 16
NW = NC * NS
ROWS_PER_W = BATCH // NW
BLK = 16
NBLK = ROWS_PER_W // BLK
NJ = FEAT // L

_mesh = plsc.VectorSubcoreMesh(core_axis_name="c", subcore_axis_name="s")


@functools.partial(
    pl.kernel,
    mesh=_mesh,
    compiler_params=pltpu.CompilerParams(needs_layout_passes=False),
    out_type=jax.ShapeDtypeStruct((BATCH * FEAT,), jnp.float32),
    scratch_types=[
        pltpu.VMEM((FEAT,), jnp.int32),
        pltpu.VMEM((BLK * FEAT,), jnp.float32),
        pltpu.VMEM((BLK * FEAT,), jnp.float32),
    ],
)
def _permute_sc(in_hbm, p_hbm, out_hbm, p_v, in_v, out_v):
    wid = lax.axis_index("s") * NC + lax.axis_index("c")
    base = wid * (ROWS_PER_W * FEAT)
    pltpu.sync_copy(p_hbm, p_v)

    def blk_body(b, carry):
        off = base + b * (BLK * FEAT)
        pltpu.sync_copy(in_hbm.at[pl.ds(off, BLK * FEAT)], in_v)

        def j_body(j, c):
            pj = p_v[pl.ds(j * L, L)]

            def r_body(r, c2):
                vals = plsc.load_gather(in_v, [pj + r * FEAT])
                out_v[pl.ds(r * FEAT + j * L, L)] = vals
                return c2

            return lax.fori_loop(0, BLK, r_body, c)

        lax.fori_loop(0, NJ, j_body, 0)
        pltpu.sync_copy(out_v, out_hbm.at[pl.ds(off, BLK * FEAT)])
        return carry

    lax.fori_loop(0, NBLK, blk_body, 0)


def kernel(inputs, p):
    flat = _permute_sc(inputs.reshape(-1), p)
    return flat.reshape(BATCH, FEAT)


# speedup vs baseline: 1.9007x; 1.9007x over previous
import functools

import jax
import jax.numpy as jnp
from jax import lax
from jax.experimental import pallas as pl
from jax.experimental.pallas import tpu as pltpu
from jax.experimental.pallas import tpu_sc as plsc

BATCH = 16384
FEAT = 2048
L = 16
NC, NS = 2, 16
NW = NC * NS
ROWS_PER_W = BATCH // NW
BLK = 8
BLKF = BLK * FEAT
NBLK = ROWS_PER_W // BLK
NJ = FEAT // L

_mesh = plsc.VectorSubcoreMesh(core_axis_name="c", subcore_axis_name="s")


@functools.partial(
    pl.kernel,
    mesh=_mesh,
    compiler_params=pltpu.CompilerParams(needs_layout_passes=False),
    out_type=jax.ShapeDtypeStruct((BATCH * FEAT,), jnp.float32),
    scratch_types=[
        pltpu.VMEM((FEAT,), jnp.int32),
        pltpu.VMEM((BLKF,), jnp.float32),
        pltpu.VMEM((BLKF,), jnp.float32),
        pltpu.VMEM((BLKF,), jnp.float32),
        pltpu.VMEM((BLKF,), jnp.float32),
        pltpu.SemaphoreType.DMA,
        pltpu.SemaphoreType.DMA,
        pltpu.SemaphoreType.DMA,
        pltpu.SemaphoreType.DMA,
    ],
)
def _permute_sc(in_hbm, p_hbm, out_hbm, p_v, in0, in1, out0, out1,
                isem0, isem1, osem0, osem1):
    wid = lax.axis_index("s") * NC + lax.axis_index("c")
    base = wid * (ROWS_PER_W * FEAT)
    pltpu.sync_copy(p_hbm, p_v)

    ins = (in0, in1)
    outs = (out0, out1)
    isems = (isem0, isem1)
    osems = (osem0, osem1)

    def in_copy(b, s):
        return pltpu.make_async_copy(
            in_hbm.at[pl.ds(base + b * BLKF, BLKF)], ins[s], isems[s])

    def out_copy(b, s):
        return pltpu.make_async_copy(
            outs[s], out_hbm.at[pl.ds(base + b * BLKF, BLKF)], osems[s])

    in_copy(0, 0).start()

    @pl.loop(0, NBLK, step=2)
    def _blocks(bb):
        for s in range(2):
            b = bb + s

            @pl.when(b + 1 < NBLK)
            def _():
                in_copy(b + 1, 1 - s).start()

            in_copy(b, s).wait()

            @pl.when(b >= 2)
            def _():
                out_copy(b - 2, s).wait()

            @plsc.parallel_loop(0, NJ, unroll=2)
            def _groups(j):
                pj = p_v[pl.ds(j * L, L)]
                for r in range(BLK):
                    vals = plsc.load_gather(ins[s], [pj + (r * FEAT)])
                    outs[s][pl.ds(r * FEAT + j * L, L)] = vals

            out_copy(b, s).start()

    out_copy(NBLK - 2, 0).wait()
    out_copy(NBLK - 1, 1).wait()


def kernel(inputs, p):
    flat = _permute_sc(inputs.reshape(-1), p)
    return flat.reshape(BATCH, FEAT)


# speedup vs baseline: 5.5799x; 2.9357x over previous
import functools

import jax
import jax.numpy as jnp
from jax import lax
from jax.experimental import pallas as pl
from jax.experimental.pallas import tpu as pltpu
from jax.experimental.pallas import tpu_sc as plsc

BATCH = 16384
FEAT = 2048
L = 16
NC, NS = 2, 16
NW = NC * NS
ROWS_PER_W = BATCH // NW
BLK = 8
NBLK = ROWS_PER_W // BLK
NJ = FEAT // L

_mesh = plsc.VectorSubcoreMesh(core_axis_name="c", subcore_axis_name="s")


@functools.partial(
    pl.kernel,
    mesh=_mesh,
    compiler_params=pltpu.CompilerParams(needs_layout_passes=False),
    out_type=jax.ShapeDtypeStruct((BATCH, FEAT), jnp.float32),
    scratch_types=[
        pltpu.VMEM((FEAT,), jnp.int32),
        pltpu.VMEM((BLK, FEAT), jnp.float32),
        pltpu.VMEM((BLK, FEAT), jnp.float32),
        pltpu.VMEM((BLK, FEAT), jnp.float32),
        pltpu.VMEM((BLK, FEAT), jnp.float32),
        pltpu.SemaphoreType.DMA,
        pltpu.SemaphoreType.DMA,
        pltpu.SemaphoreType.DMA,
        pltpu.SemaphoreType.DMA,
    ],
)
def _permute_sc(in_hbm, p_hbm, out_hbm, p_v, in0, in1, out0, out1,
                isem0, isem1, osem0, osem1):
    wid = lax.axis_index("s") * NC + lax.axis_index("c")
    base = wid * ROWS_PER_W
    pltpu.sync_copy(p_hbm, p_v)

    ins = (in0, in1)
    outs = (out0, out1)
    isems = (isem0, isem1)
    osems = (osem0, osem1)

    def in_copy(b, s):
        return pltpu.make_async_copy(
            in_hbm.at[pl.ds(base + b * BLK, BLK)], ins[s], isems[s])

    def out_copy(b, s):
        return pltpu.make_async_copy(
            outs[s], out_hbm.at[pl.ds(base + b * BLK, BLK)], osems[s])

    in_copy(0, 0).start()

    @pl.loop(0, NBLK, step=2)
    def _blocks(bb):
        for s in range(2):
            b = bb + s

            @pl.when(b + 1 < NBLK)
            def _():
                in_copy(b + 1, 1 - s).start()

            in_copy(b, s).wait()

            @pl.when(b >= 2)
            def _():
                out_copy(b - 2, s).wait()

            @plsc.parallel_loop(0, NJ, unroll=2)
            def _groups(j):
                pj = p_v[pl.ds(j * L, L)]
                for r in range(BLK):
                    rows = jnp.full((L,), r, jnp.int32)
                    vals = plsc.load_gather(ins[s], [rows, pj])
                    outs[s][r, pl.ds(j * L, L)] = vals

            out_copy(b, s).start()

    out_copy(NBLK - 2, 0).wait()
    out_copy(NBLK - 1, 1).wait()


def kernel(inputs, p):
    return _permute_sc(inputs, p)


# speedup vs baseline: 5.5846x; 1.0008x over previous
import functools

import jax
import jax.numpy as jnp
from jax import lax
from jax.experimental import pallas as pl
from jax.experimental.pallas import tpu as pltpu
from jax.experimental.pallas import tpu_sc as plsc

BATCH = 16384
FEAT = 2048
L = 16
NC, NS = 2, 16
NW = NC * NS
ROWS_PER_W = BATCH // NW
BLK = 8
NBLK = ROWS_PER_W // BLK
NJ = FEAT // L

_mesh = plsc.VectorSubcoreMesh(core_axis_name="c", subcore_axis_name="s")


@functools.partial(
    pl.kernel,
    mesh=_mesh,
    compiler_params=pltpu.CompilerParams(needs_layout_passes=False),
    out_type=jax.ShapeDtypeStruct((BATCH, FEAT), jnp.float32),
    scratch_types=[
        pltpu.VMEM((FEAT,), jnp.int32),
        pltpu.VMEM((BLK, FEAT), jnp.float32),
        pltpu.VMEM((BLK, FEAT), jnp.float32),
        pltpu.VMEM((BLK, FEAT), jnp.float32),
        pltpu.VMEM((BLK, FEAT), jnp.float32),
        pltpu.SemaphoreType.DMA,
        pltpu.SemaphoreType.DMA,
        pltpu.SemaphoreType.DMA,
        pltpu.SemaphoreType.DMA,
    ],
)
def _permute_sc(in_hbm, p_hbm, out_hbm, p_v, in0, in1, out0, out1,
                isem0, isem1, osem0, osem1):
    wid = lax.axis_index("s") * NC + lax.axis_index("c")
    base = wid * ROWS_PER_W
    pltpu.sync_copy(p_hbm, p_v)

    ins = (in0, in1)
    outs = (out0, out1)
    isems = (isem0, isem1)
    osems = (osem0, osem1)

    def in_copy(b, s):
        return pltpu.make_async_copy(
            in_hbm.at[pl.ds(base + b * BLK, BLK)], ins[s], isems[s])

    def out_copy(b, s):
        return pltpu.make_async_copy(
            outs[s], out_hbm.at[pl.ds(base + b * BLK, BLK)], osems[s])

    in_copy(0, 0).start()

    @pl.loop(0, NBLK, step=2)
    def _blocks(bb):
        for s in range(2):
            b = bb + s

            @pl.when(b + 1 < NBLK)
            def _():
                in_copy(b + 1, 1 - s).start()

            in_copy(b, s).wait()

            @pl.when(b >= 2)
            def _():
                out_copy(b - 2, s).wait()

            @plsc.parallel_loop(0, NJ, unroll=4)
            def _groups(j):
                pj = p_v[pl.ds(j * L, L)]
                for r in range(BLK):
                    rows = jnp.full((L,), r, jnp.int32)
                    vals = plsc.load_gather(ins[s], [rows, pj])
                    outs[s][r, pl.ds(j * L, L)] = vals

            out_copy(b, s).start()

    out_copy(NBLK - 2, 0).wait()
    out_copy(NBLK - 1, 1).wait()


def kernel(inputs, p):
    return _permute_sc(inputs, p)


# speedup vs baseline: 5.6926x; 1.0193x over previous
import functools

import jax
import jax.numpy as jnp
from jax import lax
from jax.experimental import pallas as pl
from jax.experimental.pallas import tpu as pltpu
from jax.experimental.pallas import tpu_sc as plsc

BATCH = 16384
FEAT = 2048
L = 16
NC, NS = 2, 16
NW = NC * NS
ROWS_PER_W = BATCH // NW
IBLK = 16
OBLK = 8
NBLK = ROWS_PER_W // IBLK
NJ = FEAT // L

_mesh = plsc.VectorSubcoreMesh(core_axis_name="c", subcore_axis_name="s")


@functools.partial(
    pl.kernel,
    mesh=_mesh,
    compiler_params=pltpu.CompilerParams(needs_layout_passes=False),
    out_type=jax.ShapeDtypeStruct((BATCH, FEAT), jnp.float32),
    scratch_types=[
        pltpu.VMEM((FEAT,), jnp.int32),
        pltpu.VMEM((IBLK, FEAT), jnp.float32),
        pltpu.VMEM((IBLK, FEAT), jnp.float32),
        pltpu.VMEM((OBLK, FEAT), jnp.float32),
        pltpu.VMEM((OBLK, FEAT), jnp.float32),
        pltpu.SemaphoreType.DMA,
        pltpu.SemaphoreType.DMA,
        pltpu.SemaphoreType.DMA,
        pltpu.SemaphoreType.DMA,
    ],
)
def _permute_sc(in_hbm, p_hbm, out_hbm, p_v, in0, in1, out0, out1,
                isem0, isem1, osem0, osem1):
    wid = lax.axis_index("s") * NC + lax.axis_index("c")
    base = wid * ROWS_PER_W
    pltpu.sync_copy(p_hbm, p_v)

    ins = (in0, in1)
    outs = (out0, out1)
    isems = (isem0, isem1)
    osems = (osem0, osem1)

    def in_copy(b, s):
        return pltpu.make_async_copy(
            in_hbm.at[pl.ds(base + b * IBLK, IBLK)], ins[s], isems[s])

    def out_copy(b, h):
        return pltpu.make_async_copy(
            outs[h],
            out_hbm.at[pl.ds(base + b * IBLK + h * OBLK, OBLK)], osems[h])

    def permute_half(in_ref, out_ref, h):
        @plsc.parallel_loop(0, NJ, unroll=4)
        def _groups(j):
            pj = p_v[pl.ds(j * L, L)]
            for r in range(OBLK):
                rows = jnp.full((L,), h * OBLK + r, jnp.int32)
                vals = plsc.load_gather(in_ref, [rows, pj])
                out_ref[r, pl.ds(j * L, L)] = vals

    in_copy(0, 0).start()

    @pl.loop(0, NBLK, step=2)
    def _blocks(bb):
        for s in range(2):
            b = bb + s

            @pl.when(b + 1 < NBLK)
            def _():
                in_copy(b + 1, 1 - s).start()

            in_copy(b, s).wait()

            for h in range(2):
                @pl.when(b >= 1)
                def _():
                    out_copy(b - 1, h).wait()

                permute_half(ins[s], outs[h], h)
                out_copy(b, h).start()

    out_copy(NBLK - 1, 0).wait()
    out_copy(NBLK - 1, 1).wait()


def kernel(inputs, p):
    return _permute_sc(inputs, p)


# speedup vs baseline: 5.7672x; 1.0131x over previous
import functools

import jax
import jax.numpy as jnp
from jax import lax
from jax.experimental import pallas as pl
from jax.experimental.pallas import tpu as pltpu
from jax.experimental.pallas import tpu_sc as plsc

BATCH = 16384
FEAT = 2048
L = 16
NC, NS = 2, 16
NW = NC * NS
ROWS_PER_W = BATCH // NW
IBLK = 16
OBLK = 8
NBLK = ROWS_PER_W // IBLK
NJ = FEAT // L

_mesh = plsc.VectorSubcoreMesh(core_axis_name="c", subcore_axis_name="s")


@functools.partial(
    pl.kernel,
    mesh=_mesh,
    compiler_params=pltpu.CompilerParams(needs_layout_passes=False),
    out_type=jax.ShapeDtypeStruct((BATCH, FEAT), jnp.float32),
    scratch_types=[
        pltpu.VMEM((FEAT,), jnp.int32),
        pltpu.VMEM((IBLK, FEAT), jnp.float32),
        pltpu.VMEM((IBLK, FEAT), jnp.float32),
        pltpu.VMEM((OBLK, FEAT), jnp.float32),
        pltpu.VMEM((OBLK, FEAT), jnp.float32),
        pltpu.SemaphoreType.DMA,
        pltpu.SemaphoreType.DMA,
        pltpu.SemaphoreType.DMA,
        pltpu.SemaphoreType.DMA,
        pltpu.SemaphoreType.DMA,
    ],
)
def _permute_sc(in_hbm, p_hbm, out_hbm, p_v, in0, in1, out0, out1,
                isem0, isem1, osem0, osem1, psem):
    wid = lax.axis_index("s") * NC + lax.axis_index("c")
    base = wid * ROWS_PER_W

    ins = (in0, in1)
    outs = (out0, out1)
    isems = (isem0, isem1)
    osems = (osem0, osem1)

    def in_copy(b, s):
        return pltpu.make_async_copy(
            in_hbm.at[pl.ds(base + b * IBLK, IBLK)], ins[s], isems[s])

    def out_copy(b, h):
        return pltpu.make_async_copy(
            outs[h],
            out_hbm.at[pl.ds(base + b * IBLK + h * OBLK, OBLK)], osems[h])

    p_dma = pltpu.make_async_copy(p_hbm, p_v, psem)
    p_dma.start()

    def permute_half(in_ref, out_ref, h):
        @plsc.parallel_loop(0, NJ, unroll=4)
        def _groups(j):
            pj = p_v[pl.ds(j * L, L)]
            for r in range(OBLK):
                rows = jnp.full((L,), h * OBLK + r, jnp.int32)
                vals = plsc.load_gather(in_ref, [rows, pj])
                out_ref[r, pl.ds(j * L, L)] = vals

    in_copy(0, 0).start()
    in_copy(1, 1).start()
    p_dma.wait()

    @pl.loop(0, NBLK, step=2)
    def _blocks(bb):
        for s in range(2):
            b = bb + s
            in_copy(b, s).wait()

            for h in range(2):
                @pl.when(b >= 1)
                def _():
                    out_copy(b - 1, h).wait()

                permute_half(ins[s], outs[h], h)
                out_copy(b, h).start()

            @pl.when(b + 2 < NBLK)
            def _():
                in_copy(b + 2, s).start()

    out_copy(NBLK - 1, 0).wait()
    out_copy(NBLK - 1, 1).wait()


def kernel(inputs, p):
    return _permute_sc(inputs, p)
